# baseline (speedup 1.0000x reference)
import functools

import jax
import jax.numpy as jnp
from jax import lax
from jax.experimental import pallas as pl
from jax.experimental.pallas import tpu as pltpu
from jax.experimental.pallas import tpu_sc as plsc

B = 16384
D = 64
NC = 2
NS = 16
NW = NC * NS
BPW = B // NW
P = 8
PB = B // P
BPWP = PB // NW


def _sc_gather2(uid, iid, ue, ie):
    mesh = plsc.VectorSubcoreMesh(core_axis_name="c", subcore_axis_name="s")

    @functools.partial(
        pl.kernel,
        mesh=mesh,
        out_type=[jax.ShapeDtypeStruct((PB, D), jnp.float32) for _ in range(2)],
        scratch_types=[
            pltpu.VMEM((BPWP,), jnp.int32),
            pltpu.VMEM((BPWP, D), jnp.float32),
            pltpu.SemaphoreType.DMA,
        ],
        compiler_params=pltpu.CompilerParams(
            use_tc_tiling_on_sc=True, needs_layout_passes=False),
        cost_estimate=pl.CostEstimate(
            flops=1_000_000, transcendentals=0, bytes_accessed=2_000_000_000),
    )
    def k(uid_h, iid_h, ue_h, ie_h, u_o, i_o, idx_v, rows_v, sem):
        wid = lax.axis_index("s") * NC + lax.axis_index("c")
        base = wid * BPWP
        lane16 = lax.iota(jnp.int32, 16)

        for ids_h, tab_h, out_h in ((uid_h, ue_h, u_o), (iid_h, ie_h, i_o)):
            pltpu.sync_copy(ids_h.at[pl.ds(base, BPWP)], idx_v)

            def fire(g):
                v = idx_v[pl.ds(g * 16, 16)]
                for j in range(16):
                    row = jnp.sum(jnp.where(lane16 == j, v, 0))
                    pltpu.async_copy(tab_h.at[row], rows_v.at[g * 16 + j], sem)

            def drain(g):
                pltpu.make_async_copy(
                    tab_h.at[pl.ds(0, 16)],
                    rows_v.at[pl.ds(g * 16, 16)], sem).wait()

            fire(0)

            def group_body(g):
                @pl.when(g + 1 < BPWP // 16)
                def _():
                    fire(g + 1)
                drain(g)

            pl.loop(0, BPWP // 16)(group_body)
            pltpu.sync_copy(rows_v, out_h.at[pl.ds(base, BPWP)])

    return k(uid, iid, ue, ie)


TCC = 2048
FIRE = 256


def _tc_d0_body(uid_r, iid_r, ue_r, ie_r, s_r, o_r,
                uids_s, iids_s, urows_v, irows_v, sem_i, sem_u, sem_v):
    pltpu.make_async_copy(uid_r.at[pl.ds(0, TCC)], uids_s, sem_i).start()
    pltpu.make_async_copy(iid_r.at[pl.ds(0, TCC)], iids_s, sem_i).start()
    pltpu.make_async_copy(uid_r.at[pl.ds(0, TCC)], uids_s, sem_i).wait()
    pltpu.make_async_copy(iid_r.at[pl.ds(0, TCC)], iids_s, sem_i).wait()

    def burst(b):
        def fire(j):
            k = b * FIRE + j
            pltpu.make_async_copy(
                ue_r.at[uids_s[k]], urows_v.at[k], sem_u).start()
            pltpu.make_async_copy(
                ie_r.at[iids_s[k]], irows_v.at[k], sem_v).start()
        pl.loop(0, FIRE, unroll=8)(fire)

        def drain(j):
            k = b * FIRE + j
            pltpu.make_async_copy(ue_r.at[0], urows_v.at[k], sem_u).wait()
            pltpu.make_async_copy(ie_r.at[0], irows_v.at[k], sem_v).wait()
        pl.loop(0, FIRE, unroll=8)(drain)

    pl.loop(0, TCC // FIRE)(burst)

    p = jnp.dot(urows_v[...], s_r[...], preferred_element_type=jnp.float32)
    o_r[...] = jnp.sum(p * irows_v[...], axis=1, keepdims=True)


def _tc_domain0(uid0, iid0, ue0, ie0, s_0):
    nb = PB // TCC
    return pl.pallas_call(
        _tc_d0_body,
        grid=(nb,),
        in_specs=[
            pl.BlockSpec(memory_space=pl.ANY),
            pl.BlockSpec(memory_space=pl.ANY),
            pl.BlockSpec(memory_space=pl.ANY),
            pl.BlockSpec(memory_space=pl.ANY),
            pl.BlockSpec((D, D), lambda i: (0, 0)),
        ],
        out_specs=pl.BlockSpec((TCC, 1), lambda i: (i, 0)),
        out_shape=jax.ShapeDtypeStruct((PB, 1), jnp.float32),
        scratch_shapes=[
            pltpu.SMEM((TCC,), jnp.int32),
            pltpu.SMEM((TCC,), jnp.int32),
            pltpu.VMEM((TCC, D), jnp.float32),
            pltpu.VMEM((TCC, D), jnp.float32),
            pltpu.SemaphoreType.DMA,
            pltpu.SemaphoreType.DMA,
            pltpu.SemaphoreType.DMA,
        ],
    )(uid0, iid0, ue0, ie0, s_0)


def _tc_body(u0_r, i0_r, u1_r, i1_r, s0_r, s1_r, o0_r, o1_r):
    p0 = jnp.dot(u0_r[...], s0_r[...], preferred_element_type=jnp.float32)
    o0_r[...] = jnp.sum(p0 * i0_r[...], axis=1, keepdims=True)
    p1 = jnp.dot(u1_r[...], s1_r[...], preferred_element_type=jnp.float32)
    o1_r[...] = jnp.sum(p1 * i1_r[...], axis=1, keepdims=True)


def _tc_dense(u0, i0, u1, i1, s_0, s_1):
    R = 2048
    nb = B // R
    row_spec = pl.BlockSpec((R, D), lambda i: (i, 0))
    s_spec = pl.BlockSpec((D, D), lambda i: (0, 0))
    out_spec = pl.BlockSpec((R, 1), lambda i: (i, 0))
    return pl.pallas_call(
        _tc_body,
        grid=(nb,),
        in_specs=[row_spec, row_spec, row_spec, row_spec, s_spec, s_spec],
        out_specs=[out_spec, out_spec],
        out_shape=[jax.ShapeDtypeStruct((B, 1), jnp.float32) for _ in range(2)],
    )(u0, i0, u1, i1, s_0, s_1)


def _tc_d1_body(u1_r, i1_r, s1_r, o1_r):
    p1 = jnp.dot(u1_r[...], s1_r[...], preferred_element_type=jnp.float32)
    o1_r[...] = jnp.sum(p1 * i1_r[...], axis=1, keepdims=True)


def _tc_dense1(u1, i1, s_1):
    R = 2048
    nb = B // R
    row_spec = pl.BlockSpec((R, D), lambda i: (i, 0))
    return pl.pallas_call(
        _tc_d1_body,
        grid=(nb,),
        in_specs=[row_spec, row_spec, pl.BlockSpec((D, D), lambda i: (0, 0))],
        out_specs=pl.BlockSpec((R, 1), lambda i: (i, 0)),
        out_shape=jax.ShapeDtypeStruct((B, 1), jnp.float32),
    )(u1, i1, s_1)


def kernel(user_ids_0, item_ids_0, user_ids_1, item_ids_1,
           user_emb_0, user_emb_1, item_emb_0, item_emb_1,
           S0, St_0, St_1):
    s_0 = jnp.concatenate([S0, St_0], axis=1)
    s_1 = jnp.concatenate([S0, St_1], axis=1)
    u1s, i1s, o0s = [], [], []
    for p in range(P):
        sl = slice(p * PB, (p + 1) * PB)
        up, ip_ = _sc_gather2(user_ids_1[sl], item_ids_1[sl],
                              user_emb_1, item_emb_1)
        u1s.append(up)
        i1s.append(ip_)
        o0s.append(_tc_domain0(user_ids_0[sl], item_ids_0[sl],
                               user_emb_0, item_emb_0, s_0))
    u1 = jnp.concatenate(u1s, axis=0)
    i1 = jnp.concatenate(i1s, axis=0)
    o0 = jnp.concatenate(o0s, axis=0)
    o1 = _tc_dense1(u1, i1, s_1)
    return jnp.concatenate([o0.reshape(1, B), o1.reshape(1, B)], axis=0)


# speedup vs baseline: 1.1561x; 1.1561x over previous
import functools

import jax
import jax.numpy as jnp
from jax import lax
from jax.experimental import pallas as pl
from jax.experimental.pallas import tpu as pltpu
from jax.experimental.pallas import tpu_sc as plsc

B = 16384
D = 64
NC = 2
NS = 16
NW = NC * NS
BPW = B // NW
P = 8
PB = B // P
BPWP = PB // NW


def _sc_gather2(uid, iid, ue, ie):
    mesh = plsc.VectorSubcoreMesh(core_axis_name="c", subcore_axis_name="s")

    @functools.partial(
        pl.kernel,
        mesh=mesh,
        out_type=[jax.ShapeDtypeStruct((B, D), jnp.float32) for _ in range(2)],
        scratch_types=[
            pltpu.VMEM((BPW,), jnp.int32),
            pltpu.VMEM((BPW, D), jnp.float32),
            [pltpu.SemaphoreType.DMA for _ in range(8)],
        ],
        compiler_params=pltpu.CompilerParams(
            use_tc_tiling_on_sc=True, needs_layout_passes=False),
        cost_estimate=pl.CostEstimate(
            flops=1_000_000, transcendentals=0, bytes_accessed=2_000_000_000),
    )
    def k(uid_h, iid_h, ue_h, ie_h, u_o, i_o, idx_v, rows_v, sems):
        wid = lax.axis_index("s") * NC + lax.axis_index("c")
        base = wid * BPW
        lane16 = lax.iota(jnp.int32, 16)

        for ids_h, tab_h, out_h in ((uid_h, ue_h, u_o), (iid_h, ie_h, i_o)):
            pltpu.sync_copy(ids_h.at[pl.ds(base, BPW)], idx_v)

            def fire(g):
                v = idx_v[pl.ds(g * 16, 16)]
                for j in range(16):
                    row = jnp.sum(jnp.where(lane16 == j, v, 0))
                    pltpu.async_copy(
                        tab_h.at[row], rows_v.at[g * 16 + j], sems[j % 8])

            def drain(g):
                for k8 in range(8):
                    pltpu.make_async_copy(
                        tab_h.at[pl.ds(0, 2)],
                        rows_v.at[pl.ds(g * 16 + 2 * k8, 2)], sems[k8]).wait()

            fire(0)

            def group_body(g):
                @pl.when(g + 1 < BPW // 16)
                def _():
                    fire(g + 1)
                drain(g)

            pl.loop(0, BPW // 16)(group_body)
            pltpu.sync_copy(rows_v, out_h.at[pl.ds(base, BPW)])

    return k(uid, iid, ue, ie)


TCC = 2048
FIRE = 256


def _tc_d0_body(uid_r, iid_r, ue_r, ie_r, s_r, o_r,
                uids_s, iids_s, urows_v, irows_v, sem_i, sem_u, sem_v):
    pltpu.make_async_copy(uid_r.at[pl.ds(0, TCC)], uids_s, sem_i).start()
    pltpu.make_async_copy(iid_r.at[pl.ds(0, TCC)], iids_s, sem_i).start()
    pltpu.make_async_copy(uid_r.at[pl.ds(0, TCC)], uids_s, sem_i).wait()
    pltpu.make_async_copy(iid_r.at[pl.ds(0, TCC)], iids_s, sem_i).wait()

    def burst(b):
        def fire(j):
            k = b * FIRE + j
            pltpu.make_async_copy(
                ue_r.at[uids_s[k]], urows_v.at[k], sem_u).start()
            pltpu.make_async_copy(
                ie_r.at[iids_s[k]], irows_v.at[k], sem_v).start()
        pl.loop(0, FIRE, unroll=8)(fire)

        def drain(j):
            k = b * FIRE + j
            pltpu.make_async_copy(ue_r.at[0], urows_v.at[k], sem_u).wait()
            pltpu.make_async_copy(ie_r.at[0], irows_v.at[k], sem_v).wait()
        pl.loop(0, FIRE, unroll=8)(drain)

    pl.loop(0, TCC // FIRE)(burst)

    p = jnp.dot(urows_v[...], s_r[...], preferred_element_type=jnp.float32)
    o_r[...] = jnp.sum(p * irows_v[...], axis=1, keepdims=True)


def _tc_domain0(uid0, iid0, ue0, ie0, s_0):
    nb = PB // TCC
    return pl.pallas_call(
        _tc_d0_body,
        grid=(nb,),
        in_specs=[
            pl.BlockSpec(memory_space=pl.ANY),
            pl.BlockSpec(memory_space=pl.ANY),
            pl.BlockSpec(memory_space=pl.ANY),
            pl.BlockSpec(memory_space=pl.ANY),
            pl.BlockSpec((D, D), lambda i: (0, 0)),
        ],
        out_specs=pl.BlockSpec((TCC, 1), lambda i: (i, 0)),
        out_shape=jax.ShapeDtypeStruct((PB, 1), jnp.float32),
        scratch_shapes=[
            pltpu.SMEM((TCC,), jnp.int32),
            pltpu.SMEM((TCC,), jnp.int32),
            pltpu.VMEM((TCC, D), jnp.float32),
            pltpu.VMEM((TCC, D), jnp.float32),
            pltpu.SemaphoreType.DMA,
            pltpu.SemaphoreType.DMA,
            pltpu.SemaphoreType.DMA,
        ],
    )(uid0, iid0, ue0, ie0, s_0)


def _tc_body(u0_r, i0_r, u1_r, i1_r, s0_r, s1_r, o0_r, o1_r):
    p0 = jnp.dot(u0_r[...], s0_r[...], preferred_element_type=jnp.float32)
    o0_r[...] = jnp.sum(p0 * i0_r[...], axis=1, keepdims=True)
    p1 = jnp.dot(u1_r[...], s1_r[...], preferred_element_type=jnp.float32)
    o1_r[...] = jnp.sum(p1 * i1_r[...], axis=1, keepdims=True)


def _tc_dense(u0, i0, u1, i1, s_0, s_1):
    R = 2048
    nb = B // R
    row_spec = pl.BlockSpec((R, D), lambda i: (i, 0))
    s_spec = pl.BlockSpec((D, D), lambda i: (0, 0))
    out_spec = pl.BlockSpec((R, 1), lambda i: (i, 0))
    return pl.pallas_call(
        _tc_body,
        grid=(nb,),
        in_specs=[row_spec, row_spec, row_spec, row_spec, s_spec, s_spec],
        out_specs=[out_spec, out_spec],
        out_shape=[jax.ShapeDtypeStruct((B, 1), jnp.float32) for _ in range(2)],
    )(u0, i0, u1, i1, s_0, s_1)


def _tc_d1_body(u1_r, i1_r, s1_r, o1_r):
    p1 = jnp.dot(u1_r[...], s1_r[...], preferred_element_type=jnp.float32)
    o1_r[...] = jnp.sum(p1 * i1_r[...], axis=1, keepdims=True)


def _tc_dense1(u1, i1, s_1):
    R = 2048
    nb = B // R
    row_spec = pl.BlockSpec((R, D), lambda i: (i, 0))
    return pl.pallas_call(
        _tc_d1_body,
        grid=(nb,),
        in_specs=[row_spec, row_spec, pl.BlockSpec((D, D), lambda i: (0, 0))],
        out_specs=pl.BlockSpec((R, 1), lambda i: (i, 0)),
        out_shape=jax.ShapeDtypeStruct((B, 1), jnp.float32),
    )(u1, i1, s_1)


def kernel(user_ids_0, item_ids_0, user_ids_1, item_ids_1,
           user_emb_0, user_emb_1, item_emb_0, item_emb_1,
           S0, St_0, St_1):
    s_0 = jnp.concatenate([S0, St_0], axis=1)
    s_1 = jnp.concatenate([S0, St_1], axis=1)
    u0, i0 = _sc_gather2(user_ids_0, item_ids_0, user_emb_0, item_emb_0)
    u1, i1 = _sc_gather2(user_ids_1, item_ids_1, user_emb_1, item_emb_1)
    o0, o1 = _tc_dense(u0, i0, u1, i1, s_0, s_1)
    return jnp.concatenate([o0.reshape(1, B), o1.reshape(1, B)], axis=0)
